# Initial kernel scaffold; baseline (speedup 1.0000x reference)
import jax
import jax.numpy as jnp
from jax.experimental import pallas as pl


def kernel(coords, pos_embed):
    raise NotImplementedError("write your pallas kernel here")


# speedup vs baseline: 3.5966x; 3.5966x over previous
import functools

import jax
import jax.numpy as jnp
from jax import lax
from jax.experimental import pallas as pl
from jax.experimental.pallas import tpu as pltpu
from jax.experimental.pallas import tpu_sc as plsc

_H, _W, _D = 32, 32, 768
_B = 64
_EXTRA = 1
_HW = _H * _W
_T = _HW + _EXTRA

_NC, _NS = 2, 16
_NW = _NC * _NS
_BPW = _B // _NW
_CHUNK = 64
_NFULL = 15
_TAIL = 72
_LANES = 16


def _sc_body(coords_hbm, pos_hbm, out_hbm, cidx_v, idx_v, buf_v, row_v, sem):
    wid = lax.axis_index("s") * _NC + lax.axis_index("c")

    for i in range(_BPW):
        b = wid * _BPW + i

        pltpu.sync_copy(coords_hbm.at[b], cidx_v)

        idx_v[pl.ds(0, _LANES)] = jnp.zeros((_LANES,), jnp.int32)

        def idx_step(j, carry):
            lanes = lax.iota(jnp.int32, _LANES)
            pos = lanes * 2 + j * (2 * _LANES)
            r = plsc.load_gather(cidx_v, [pos])
            c = plsc.load_gather(cidx_v, [pos + 1])
            plsc.store_scatter(idx_v, [lanes + (1 + j * _LANES)],
                               r * _W + c + _EXTRA)
            return carry

        lax.fori_loop(0, _HW // _LANES, idx_step, 0)

        last = jnp.full((_LANES,), 2 * _HW - 2, jnp.int32)
        rl = plsc.load_gather(cidx_v, [last])
        cl = plsc.load_gather(cidx_v, [last + 1])
        idx_v[pl.ds(_HW, _LANES)] = rl * _W + cl + _EXTRA

        def chunk_step(j, carry):
            gather = pltpu.make_async_copy(
                pos_hbm.at[idx_v.at[pl.ds(j * _CHUNK, _CHUNK)]],
                buf_v.at[pl.ds(0, _CHUNK)], sem)
            gather.start()
            gather.wait()
            pltpu.sync_copy(buf_v.at[pl.ds(0, _CHUNK)],
                            out_hbm.at[b, pl.ds(j * _CHUNK, _CHUNK)])
            return carry

        lax.fori_loop(0, _NFULL, chunk_step, 0)

        tail = pltpu.make_async_copy(
            pos_hbm.at[idx_v.at[pl.ds(_NFULL * _CHUNK, _TAIL)]], buf_v, sem)
        tail.start()
        tail.wait()
        pltpu.sync_copy(buf_v.at[pl.ds(0, _CHUNK)],
                        out_hbm.at[b, pl.ds(_NFULL * _CHUNK, _CHUNK)])

        def row_step(t, carry):
            row_v[0, pl.ds(t * _LANES, _LANES)] = \
                buf_v[_CHUNK, pl.ds(t * _LANES, _LANES)]
            return carry

        lax.fori_loop(0, _D // _LANES, row_step, 0)
        pltpu.sync_copy(row_v, out_hbm.at[b, pl.ds(_T - 1, 1)])


@jax.jit
def _sc_call(coords2d, pos2d):
    mesh = plsc.VectorSubcoreMesh(core_axis_name="c", subcore_axis_name="s")
    return pl.kernel(
        _sc_body,
        out_type=jax.ShapeDtypeStruct((_B, _T, _D), jnp.float32),
        mesh=mesh,
        compiler_params=pltpu.CompilerParams(needs_layout_passes=False),
        scratch_types=[
            pltpu.VMEM((2 * _HW,), jnp.int32),
            pltpu.VMEM((1152,), jnp.int32),
            pltpu.VMEM((_TAIL, _D), jnp.float32),
            pltpu.VMEM((1, _D), jnp.float32),
            pltpu.SemaphoreType.DMA,
        ],
    )(coords2d, pos2d)


def kernel(coords, pos_embed):
    coords2d = coords.reshape(_B, 2 * _HW).astype(jnp.int32)
    pos2d = pos_embed.reshape(_EXTRA + _HW, _D)
    return _sc_call(coords2d, pos2d)


# speedup vs baseline: 3.7786x; 1.0506x over previous
import functools

import jax
import jax.numpy as jnp
from jax import lax
from jax.experimental import pallas as pl
from jax.experimental.pallas import tpu as pltpu
from jax.experimental.pallas import tpu_sc as plsc

_H, _W, _D = 32, 32, 768
_B = 64
_EXTRA = 1
_HW = _H * _W
_T = _HW + _EXTRA

_NC, _NS = 2, 16
_NW = _NC * _NS
_BPW = _B // _NW
_CHUNK = 64
_NFULL = 15
_TAIL = 72
_LANES = 16


def _sc_body(coords_hbm, pos_hbm, out_hbm, cidx_v, idx_v,
             buf_a, buf_b, row_v, sem_a, sem_b):
    wid = lax.axis_index("s") * _NC + lax.axis_index("c")

    def start64(j, buf, sem):
        pltpu.make_async_copy(
            pos_hbm.at[idx_v.at[pl.ds(j * _CHUNK, _CHUNK)]],
            buf.at[pl.ds(0, _CHUNK)], sem).start()

    def wait64(buf, sem):
        pltpu.make_async_copy(
            pos_hbm.at[idx_v.at[pl.ds(0, _CHUNK)]],
            buf.at[pl.ds(0, _CHUNK)], sem).wait()

    def write64(j, b, buf):
        pltpu.sync_copy(buf.at[pl.ds(0, _CHUNK)],
                        out_hbm.at[b, pl.ds(j * _CHUNK, _CHUNK)])

    for i in range(_BPW):
        b = wid * _BPW + i

        pltpu.sync_copy(coords_hbm.at[b], cidx_v)

        idx_v[pl.ds(0, _LANES)] = jnp.zeros((_LANES,), jnp.int32)

        def idx_step(j, carry):
            lanes = lax.iota(jnp.int32, _LANES)
            pos = lanes * 2 + j * (2 * _LANES)
            r = plsc.load_gather(cidx_v, [pos])
            c = plsc.load_gather(cidx_v, [pos + 1])
            plsc.store_scatter(idx_v, [lanes + (1 + j * _LANES)],
                               r * _W + c + _EXTRA)
            return carry

        lax.fori_loop(0, _HW // _LANES, idx_step, 0)

        last = jnp.full((_LANES,), 2 * _HW - 2, jnp.int32)
        rl = plsc.load_gather(cidx_v, [last])
        cl = plsc.load_gather(cidx_v, [last + 1])
        idx_v[pl.ds(_HW, _LANES)] = rl * _W + cl + _EXTRA

        start64(0, buf_a, sem_a)

        def pair_step(jj, carry):
            j0 = jj * 2
            start64(j0 + 1, buf_b, sem_b)
            wait64(buf_a, sem_a)
            write64(j0, b, buf_a)
            start64(j0 + 2, buf_a, sem_a)
            wait64(buf_b, sem_b)
            write64(j0 + 1, b, buf_b)
            return carry

        lax.fori_loop(0, (_NFULL - 1) // 2, pair_step, 0)

        tail = pltpu.make_async_copy(
            pos_hbm.at[idx_v.at[pl.ds(_NFULL * _CHUNK, _TAIL)]], buf_b, sem_b)
        tail.start()
        wait64(buf_a, sem_a)
        write64(_NFULL - 1, b, buf_a)
        tail.wait()
        write64(_NFULL, b, buf_b)

        def row_step(t, carry):
            row_v[0, pl.ds(t * _LANES, _LANES)] = \
                buf_b[_CHUNK, pl.ds(t * _LANES, _LANES)]
            return carry

        lax.fori_loop(0, _D // _LANES, row_step, 0)
        pltpu.sync_copy(row_v, out_hbm.at[b, pl.ds(_T - 1, 1)])


@jax.jit
def _sc_call(coords2d, pos2d):
    mesh = plsc.VectorSubcoreMesh(core_axis_name="c", subcore_axis_name="s")
    return pl.kernel(
        _sc_body,
        out_type=jax.ShapeDtypeStruct((_B, _T, _D), jnp.float32),
        mesh=mesh,
        compiler_params=pltpu.CompilerParams(needs_layout_passes=False),
        scratch_types=[
            pltpu.VMEM((2 * _HW,), jnp.int32),
            pltpu.VMEM((1152,), jnp.int32),
            pltpu.VMEM((_TAIL, _D), jnp.float32),
            pltpu.VMEM((_TAIL, _D), jnp.float32),
            pltpu.VMEM((1, _D), jnp.float32),
            pltpu.SemaphoreType.DMA,
            pltpu.SemaphoreType.DMA,
        ],
    )(coords2d, pos2d)


def kernel(coords, pos_embed):
    coords2d = coords.reshape(_B, 2 * _HW).astype(jnp.int32)
    pos2d = pos_embed.reshape(_EXTRA + _HW, _D)
    return _sc_call(coords2d, pos2d)
